# Initial kernel scaffold; baseline (speedup 1.0000x reference)
import jax
import jax.numpy as jnp
from jax.experimental import pallas as pl


def kernel(x_train, query, K):
    raise NotImplementedError("write your pallas kernel here")


# speedup vs baseline: 12.6360x; 12.6360x over previous
import functools

import jax
import jax.numpy as jnp
from jax import lax
from jax.experimental import pallas as pl
from jax.experimental.pallas import tpu as pltpu
from jax.experimental.pallas import tpu_sc as plsc

_N = 100000
_D = 128
_Q = 1024
_NB = 512
_NPAD = ((_N + _NB - 1) // _NB) * _NB


def _keys_kernel(q_ref, x_ref, qn_ref, xn_ref, out_ref):
    qx = jnp.dot(q_ref[...], x_ref[...].T, preferred_element_type=jnp.float32)
    d = (qn_ref[...] + xn_ref[...]) - 2.0 * qx
    u = jax.lax.bitcast_convert_type(d, jnp.int32)
    au = u & 0x7FFFFFFF
    r = au + 0xFFF + ((au >> 13) & 1)
    h = jnp.maximum((r - 0x38000000) >> 13, 0)
    out_ref[...] = 0x8000 + (h ^ (u >> 31))


def _compute_keys(q32, x32, qn32, xn32):
    grid = (_NPAD // _NB,)
    return pl.pallas_call(
        _keys_kernel,
        grid=grid,
        in_specs=[
            pl.BlockSpec((_Q, _D), lambda i: (0, 0)),
            pl.BlockSpec((_NB, _D), lambda i: (i, 0)),
            pl.BlockSpec((_Q, 1), lambda i: (0, 0)),
            pl.BlockSpec((1, _NB), lambda i: (0, i)),
        ],
        out_specs=pl.BlockSpec((_Q, _NB), lambda i: (0, i)),
        out_shape=jax.ShapeDtypeStruct((_Q, _NPAD), jnp.int32),
    )(q32, x32, qn32, xn32)


_C = _NPAD // 4
_NV = _C // 16
_QPW = _Q // 32


def _gather16(x, idx):
    dnums = lax.GatherDimensionNumbers(
        offset_dims=(), collapsed_slice_dims=(0,), start_index_map=(0,))
    return lax.gather(x, idx[:, None], dnums, (1,),
                      mode=lax.GatherScatterMode.PROMISE_IN_BOUNDS)


def _sc_select(keys):
    mesh = plsc.VectorSubcoreMesh(core_axis_name="c", subcore_axis_name="s")

    @functools.partial(
        pl.kernel,
        mesh=mesh,
        out_type=jax.ShapeDtypeStruct((_Q, 32), jnp.int32),
        compiler_params=pltpu.CompilerParams(needs_layout_passes=False),
        scratch_types=[
            pltpu.VMEM((_C,), jnp.int32),
            pltpu.VMEM((_C,), jnp.int32),
            pltpu.VMEM((32,), jnp.int32),
            pltpu.SemaphoreType.DMA,
            pltpu.SemaphoreType.DMA,
        ],
    )
    def sel(keys_hbm, out_hbm, buf0, buf1, outb, sem0, sem1):
        wid = lax.axis_index("s") * 2 + lax.axis_index("c")
        iota = lax.iota(jnp.int32, 16)
        INF = jnp.int32(0x7FFFFFFF)
        full15 = jnp.full((16,), 15, jnp.int32)
        bufs = (buf0, buf1)
        sems = (sem0, sem1)

        def scal(x):
            return jnp.squeeze(lax.slice(x, (0,), (1,)))

        def scan_chunk(carry, buf, base_int):
            def vbody(i, c2):
                L0, L1, I0, I1, t, gidx = c2
                v = buf[pl.ds(i * 16, 16)]
                m = v < t

                def do_insert(st):
                    def wcond(st2):
                        return scal(
                            plsc.all_reduce_population_count(st2[5])) > 0

                    def wbody(st2):
                        L0, L1, I0, I1, t, m2 = st2
                        lanev = plsc.all_reduce_ffs(m2)
                        kv = _gather16(v, lanev)
                        giv = _gather16(gidx, lanev)
                        pos = (plsc.all_reduce_population_count(L0 <= kv)
                               + plsc.all_reduce_population_count(L1 <= kv))
                        sh = jnp.maximum(iota - 1, 0)
                        L0s = _gather16(L0, sh)
                        L1s = _gather16(L1, sh)
                        I0s = _gather16(I0, sh)
                        I1s = _gather16(I1, sh)
                        l0last = _gather16(L0, full15)
                        i0last = _gather16(I0, full15)
                        g1 = iota + 16
                        nL0 = jnp.where(iota < pos, L0,
                                        jnp.where(iota == pos, kv, L0s))
                        nI0 = jnp.where(iota < pos, I0,
                                        jnp.where(iota == pos, giv, I0s))
                        nL1 = jnp.where(
                            g1 < pos, L1,
                            jnp.where(g1 == pos, kv,
                                      jnp.where(iota == 0, l0last, L1s)))
                        nI1 = jnp.where(
                            g1 < pos, I1,
                            jnp.where(g1 == pos, giv,
                                      jnp.where(iota == 0, i0last, I1s)))
                        nt = _gather16(nL1, full15)
                        nm = m2 & (iota != lanev) & (v < nt)
                        return (nL0, nL1, nI0, nI1, nt, nm)

                    L0, L1, I0, I1, t, _m = lax.while_loop(
                        wcond, wbody, st)
                    return (L0, L1, I0, I1, t)

                L0, L1, I0, I1, t = lax.cond(
                    scal(plsc.all_reduce_population_count(m)) > 0,
                    do_insert, lambda st: st[:5],
                    (L0, L1, I0, I1, t, m))
                return (L0, L1, I0, I1, t, gidx + 16)

            st = carry + (iota + base_int,)
            return lax.fori_loop(0, _NV, vbody, st)[:5]

        def qbody(qi, _):
            q = wid * _QPW + qi
            row = keys_hbm.at[q]
            carry = (jnp.full((16,), INF), jnp.full((16,), INF),
                     jnp.zeros((16,), jnp.int32), jnp.zeros((16,), jnp.int32),
                     jnp.full((16,), INF))
            nxt = pltpu.async_copy(row.at[pl.ds(0, _C)], buf0, sem0)
            for c in range(4):
                cur = nxt
                if c + 1 < 4:
                    nxt = pltpu.async_copy(
                        row.at[pl.ds((c + 1) * _C, _C)],
                        bufs[(c + 1) % 2], sems[(c + 1) % 2])
                cur.wait()
                carry = scan_chunk(carry, bufs[c % 2], c * _C)
            L0, L1, I0, I1, _t = carry
            outb[pl.ds(0, 16)] = I0
            outb[pl.ds(16, 16)] = I1
            pltpu.sync_copy(outb, out_hbm.at[q])
            return 0

        lax.fori_loop(0, _QPW, qbody, 0)

    return sel(keys)


def kernel(x_train, query, K):
    xt16 = x_train.astype(jnp.float16)
    q16 = query.astype(jnp.float16)
    xn16 = (xt16 ** 2).sum(-1)
    qn16 = (q16 ** 2).sum(-1)
    x32 = jnp.pad(xt16.astype(jnp.float32), ((0, _NPAD - _N), (0, 0)))
    xn32 = jnp.concatenate(
        [xn16.astype(jnp.float32),
         jnp.full((_NPAD - _N,), 3.0e38, jnp.float32)])
    q32 = q16.astype(jnp.float32)
    keys = _compute_keys(q32, x32, qn16.astype(jnp.float32)[:, None],
                         xn32[None, :])
    return _sc_select(keys)
